# Initial kernel scaffold; baseline (speedup 1.0000x reference)
import jax
import jax.numpy as jnp
from jax.experimental import pallas as pl


def kernel(X, E, W, bias, gamma, beta):
    raise NotImplementedError("write your pallas kernel here")


# speedup vs baseline: 1.0176x; 1.0176x over previous
import functools

import jax
import jax.numpy as jnp
from jax.experimental import pallas as pl
from jax.experimental.pallas import tpu as pltpu


def _mm_stats_kernel(x_ref, e_ref, w_ref, y_ref, sum_ref, sq_ref, xw_ref):
    i = pl.program_id(0)

    @pl.when(i == 0)
    def _init():
        xw_ref[...] = jnp.dot(x_ref[...], w_ref[...],
                              preferred_element_type=jnp.float32)
        sum_ref[...] = jnp.zeros_like(sum_ref)
        sq_ref[...] = jnp.zeros_like(sq_ref)

    y = jnp.dot(e_ref[...], xw_ref[...], preferred_element_type=jnp.float32)
    y_ref[...] = y
    sum_ref[...] += jnp.sum(y, axis=0, keepdims=True)
    sq_ref[...] += jnp.sum(y * y, axis=0, keepdims=True)


def _bn_act_kernel(n, y_ref, sum_ref, sq_ref, gamma_ref, beta_ref, o_ref):
    mean = sum_ref[...] * (1.0 / n)
    var = sq_ref[...] * (1.0 / n) - mean * mean
    inv = jax.lax.rsqrt(var + 1e-5)
    scale = gamma_ref[...] * inv
    shift = beta_ref[...] - mean * scale
    z = y_ref[...] * scale + shift
    o_ref[...] = jnp.where(z >= 0, z, 0.01 * z)


def kernel(X, E, W, bias, gamma, beta):
    del bias
    n, d_in = X.shape
    d_out = W.shape[1]
    bm = 400
    grid = n // bm

    y, colsum, colsq = pl.pallas_call(
        _mm_stats_kernel,
        grid=(grid,),
        in_specs=[
            pl.BlockSpec((n, d_in), lambda i: (0, 0)),
            pl.BlockSpec((bm, n), lambda i: (i, 0)),
            pl.BlockSpec((d_in, d_out), lambda i: (0, 0)),
        ],
        out_specs=[
            pl.BlockSpec((bm, d_out), lambda i: (i, 0)),
            pl.BlockSpec((1, d_out), lambda i: (0, 0)),
            pl.BlockSpec((1, d_out), lambda i: (0, 0)),
        ],
        out_shape=[
            jax.ShapeDtypeStruct((n, d_out), jnp.float32),
            jax.ShapeDtypeStruct((1, d_out), jnp.float32),
            jax.ShapeDtypeStruct((1, d_out), jnp.float32),
        ],
        scratch_shapes=[pltpu.VMEM((n, d_out), jnp.float32)],
    )(X, E, W)

    out = pl.pallas_call(
        functools.partial(_bn_act_kernel, float(n)),
        grid=(1,),
        in_specs=[
            pl.BlockSpec((n, d_out), lambda i: (0, 0)),
            pl.BlockSpec((1, d_out), lambda i: (0, 0)),
            pl.BlockSpec((1, d_out), lambda i: (0, 0)),
            pl.BlockSpec((1, d_out), lambda i: (0, 0)),
            pl.BlockSpec((1, d_out), lambda i: (0, 0)),
        ],
        out_specs=pl.BlockSpec((n, d_out), lambda i: (0, 0)),
        out_shape=jax.ShapeDtypeStruct((n, d_out), jnp.float32),
    )(y, colsum, colsq, gamma.reshape(1, d_out), beta.reshape(1, d_out))

    return out


# speedup vs baseline: 1.0506x; 1.0324x over previous
import functools

import jax
import jax.numpy as jnp
from jax.experimental import pallas as pl
from jax.experimental.pallas import tpu as pltpu


def _fused_kernel(x_ref, e_ref, w_ref, g_ref, b_ref, o_ref,
                  xw_ref, sum_ref, sq_ref, *, nsteps, bm, n):
    i = pl.program_id(0)

    @pl.when(i == 0)
    def _init():
        xw_ref[...] = jnp.dot(x_ref[...], w_ref[...],
                              preferred_element_type=jnp.float32)
        sum_ref[...] = jnp.zeros_like(sum_ref)
        sq_ref[...] = jnp.zeros_like(sq_ref)

    y = jnp.dot(e_ref[...], xw_ref[...], preferred_element_type=jnp.float32)
    o_ref[pl.ds(i * bm, bm), :] = y
    sum_ref[...] += jnp.sum(y, axis=0, keepdims=True)
    sq_ref[...] += jnp.sum(y * y, axis=0, keepdims=True)

    @pl.when(i == nsteps - 1)
    def _finalize():
        mean = sum_ref[...] * (1.0 / n)
        var = sq_ref[...] * (1.0 / n) - mean * mean
        scale = g_ref[...] * jax.lax.rsqrt(var + 1e-5)
        shift = b_ref[...] - mean * scale
        z = o_ref[...] * scale + shift
        o_ref[...] = jnp.where(z >= 0, z, 0.01 * z)


def kernel(X, E, W, bias, gamma, beta):
    del bias
    n, d_in = X.shape
    d_out = W.shape[1]
    bm = 400
    nsteps = n // bm

    return pl.pallas_call(
        functools.partial(_fused_kernel, nsteps=nsteps, bm=bm, n=float(n)),
        grid=(nsteps,),
        in_specs=[
            pl.BlockSpec((n, d_in), lambda i: (0, 0)),
            pl.BlockSpec((bm, n), lambda i: (i, 0)),
            pl.BlockSpec((d_in, d_out), lambda i: (0, 0)),
            pl.BlockSpec((1, d_out), lambda i: (0, 0)),
            pl.BlockSpec((1, d_out), lambda i: (0, 0)),
        ],
        out_specs=pl.BlockSpec((n, d_out), lambda i: (0, 0)),
        out_shape=jax.ShapeDtypeStruct((n, d_out), jnp.float32),
        scratch_shapes=[
            pltpu.VMEM((n, d_out), jnp.float32),
            pltpu.VMEM((1, d_out), jnp.float32),
            pltpu.VMEM((1, d_out), jnp.float32),
        ],
    )(X, E, W, gamma.reshape(1, d_out), beta.reshape(1, d_out))
